# initial kernel scaffold (unmeasured)
import jax
import jax.numpy as jnp
from jax import lax
from jax.experimental import pallas as pl
from jax.experimental.pallas import tpu as pltpu

N_DEV = 8
N_TOK = 2048
D_MODEL = 512
D_HID = 1024
N_EXP = 32
N_EXP_LOCAL = N_EXP // N_DEV
CHUNK = N_TOK // N_DEV
N_SLOTS = 2 * (N_DEV - 1) + 1


def kernel(x, router_W, route_idx, expert_W):
    def body(x_ref, rw_ref, idx_ref, ew_ref, out_ref, acc_ref, comm_ref,
             send_sems, recv_sems):
        my = lax.axis_index("i")
        left = lax.rem(my + N_DEV - 1, N_DEV)
        right = lax.rem(my + 1, N_DEV)

        barrier_sem = pltpu.get_barrier_semaphore()
        for nbr in (left, right):
            pl.semaphore_signal(
                barrier_sem, inc=1,
                device_id=(nbr,), device_id_type=pl.DeviceIdType.MESH,
            )
        pl.semaphore_wait(barrier_sem, 2)

        xf = x_ref[:, :]
        scores = jnp.dot(xf, rw_ref[:, :], preferred_element_type=jnp.float32)
        idx = idx_ref[:, :]
        e0 = idx[:, 0:1]
        e1 = idx[:, 1:2]
        iota = lax.broadcasted_iota(jnp.int32, (N_TOK, N_EXP), 1)
        s0 = jnp.sum(jnp.where(iota == e0, scores, 0.0), axis=1, keepdims=True)
        s1 = jnp.sum(jnp.where(iota == e1, scores, 0.0), axis=1, keepdims=True)
        m = jnp.maximum(s0, s1)
        g0 = jnp.exp(s0 - m)
        g1 = jnp.exp(s1 - m)
        w0 = g0 / (g0 + g1)
        w1 = g1 / (g0 + g1)

        xb = xf.astype(jnp.bfloat16)
        acc_ref[:, :] = jnp.zeros((N_TOK, D_HID), jnp.float32)
        for k in range(N_EXP_LOCAL):
            ge = my * N_EXP_LOCAL + k
            gate = (w0 * (e0 == ge).astype(jnp.float32)
                    + w1 * (e1 == ge).astype(jnp.float32))
            y = jnp.dot(xb, ew_ref[k].astype(jnp.bfloat16),
                        preferred_element_type=jnp.float32)
            acc_ref[:, :] = acc_ref[:, :] + gate * y

        comm_ref[0] = acc_ref[pl.ds(my * CHUNK, CHUNK), :].astype(jnp.bfloat16)
        for s in range(N_DEV - 1):
            rdma = pltpu.make_async_remote_copy(
                src_ref=comm_ref.at[s],
                dst_ref=comm_ref.at[s + 1],
                send_sem=send_sems.at[s],
                recv_sem=recv_sems.at[s + 1],
                device_id=(right,),
                device_id_type=pl.DeviceIdType.MESH,
            )
            rdma.start()
            rdma.wait()
            c = lax.rem(my + N_DEV - s - 1, N_DEV)
            comm_ref[s + 1] = (
                comm_ref[s + 1][:, :].astype(jnp.float32)
                + acc_ref[pl.ds(c * CHUNK, CHUNK), :]
            ).astype(jnp.bfloat16)

        own = lax.rem(my + 1, N_DEV)
        out_ref[pl.ds(own * CHUNK, CHUNK), :] = (
            comm_ref[N_DEV - 1][:, :].astype(jnp.float32))

        for s in range(N_DEV - 1):
            slot = (N_DEV - 1) + s
            rdma = pltpu.make_async_remote_copy(
                src_ref=comm_ref.at[slot],
                dst_ref=comm_ref.at[slot + 1],
                send_sem=send_sems.at[slot],
                recv_sem=recv_sems.at[slot + 1],
                device_id=(right,),
                device_id_type=pl.DeviceIdType.MESH,
            )
            rdma.start()
            rdma.wait()
            c = lax.rem(my + N_DEV - s, N_DEV)
            out_ref[pl.ds(c * CHUNK, CHUNK), :] = (
                comm_ref[slot + 1][:, :].astype(jnp.float32))

    return pl.pallas_call(
        body,
        out_shape=jax.ShapeDtypeStruct((N_TOK, D_HID), jnp.float32),
        in_specs=[
            pl.BlockSpec(memory_space=pltpu.VMEM),
            pl.BlockSpec(memory_space=pltpu.VMEM),
            pl.BlockSpec(memory_space=pltpu.VMEM),
            pl.BlockSpec(memory_space=pltpu.VMEM),
        ],
        out_specs=pl.BlockSpec(memory_space=pltpu.VMEM),
        scratch_shapes=[
            pltpu.VMEM((N_TOK, D_HID), jnp.float32),
            pltpu.VMEM((N_SLOTS, CHUNK, D_HID), jnp.bfloat16),
            pltpu.SemaphoreType.DMA((N_SLOTS,)),
            pltpu.SemaphoreType.DMA((N_SLOTS,)),
        ],
        compiler_params=pltpu.CompilerParams(collective_id=0),
    )(x, router_W, route_idx, expert_W)


# baseline (device time: 129701 ns/iter reference)
import jax
import jax.numpy as jnp
from jax import lax
from jax.experimental import pallas as pl
from jax.experimental.pallas import tpu as pltpu

N_DEV = 8
N_TOK = 2048
D_MODEL = 512
D_HID = 1024
N_EXP = 32
N_EXP_LOCAL = N_EXP // N_DEV
CHUNK = N_TOK // N_DEV
N_SLOTS = 2 * (N_DEV - 1) + 1


def kernel(x, router_W, route_idx, expert_W):
    def body(x_ref, rw_ref, idx_ref, ew_ref, out_ref, acc_ref, comm_ref,
             send_sems, recv_sems):
        my = lax.axis_index("i")
        left = lax.rem(my + N_DEV - 1, N_DEV)
        right = lax.rem(my + 1, N_DEV)

        barrier_sem = pltpu.get_barrier_semaphore()
        for nbr in (left, right):
            pl.semaphore_signal(
                barrier_sem, inc=1,
                device_id=(nbr,), device_id_type=pl.DeviceIdType.MESH,
            )
        pl.semaphore_wait(barrier_sem, 2)

        xf = x_ref[:, :]
        scores = jnp.dot(xf, rw_ref[:, :], preferred_element_type=jnp.float32)
        idx = idx_ref[:, :]
        e0 = idx[:, 0:1]
        e1 = idx[:, 1:2]
        iota = lax.broadcasted_iota(jnp.int32, (N_TOK, N_EXP), 1)
        s0 = jnp.sum(jnp.where(iota == e0, scores, 0.0), axis=1, keepdims=True)
        s1 = jnp.sum(jnp.where(iota == e1, scores, 0.0), axis=1, keepdims=True)
        m = jnp.maximum(s0, s1)
        g0 = jnp.exp(s0 - m)
        g1 = jnp.exp(s1 - m)
        w0 = g0 / (g0 + g1)
        w1 = g1 / (g0 + g1)

        xb = xf.astype(jnp.bfloat16)
        acc_ref[:, :] = jnp.zeros((N_TOK, D_HID), jnp.bfloat16)
        for k in range(N_EXP_LOCAL):
            ge = my * N_EXP_LOCAL + k
            gate = (w0 * (e0 == ge).astype(jnp.float32)
                    + w1 * (e1 == ge).astype(jnp.float32))
            y = jnp.dot(xb, ew_ref[k].astype(jnp.bfloat16),
                        preferred_element_type=jnp.float32)
            acc_ref[:, :] = (acc_ref[:, :].astype(jnp.float32)
                             + gate * y).astype(jnp.bfloat16)

        comm_ref[0] = acc_ref[pl.ds(my * CHUNK, CHUNK), :]
        for s in range(N_DEV - 1):
            rdma = pltpu.make_async_remote_copy(
                src_ref=comm_ref.at[s],
                dst_ref=comm_ref.at[s + 1],
                send_sem=send_sems.at[s],
                recv_sem=recv_sems.at[s + 1],
                device_id=(right,),
                device_id_type=pl.DeviceIdType.MESH,
            )
            rdma.start()
            rdma.wait()
            c = lax.rem(my + N_DEV - s - 1, N_DEV)
            comm_ref[s + 1] = (
                comm_ref[s + 1][:, :].astype(jnp.float32)
                + acc_ref[pl.ds(c * CHUNK, CHUNK), :].astype(jnp.float32)
            ).astype(jnp.bfloat16)

        own = lax.rem(my + 1, N_DEV)
        out_ref[pl.ds(own * CHUNK, CHUNK), :] = comm_ref[N_DEV - 1][:, :]

        for s in range(N_DEV - 1):
            slot = (N_DEV - 1) + s
            rdma = pltpu.make_async_remote_copy(
                src_ref=comm_ref.at[slot],
                dst_ref=comm_ref.at[slot + 1],
                send_sem=send_sems.at[slot],
                recv_sem=recv_sems.at[slot + 1],
                device_id=(right,),
                device_id_type=pl.DeviceIdType.MESH,
            )
            rdma.start()
            rdma.wait()
            c = lax.rem(my + N_DEV - s, N_DEV)
            out_ref[pl.ds(c * CHUNK, CHUNK), :] = comm_ref[slot + 1][:, :]

    return pl.pallas_call(
        body,
        out_shape=jax.ShapeDtypeStruct((N_TOK, D_HID), jnp.bfloat16),
        in_specs=[
            pl.BlockSpec(memory_space=pltpu.VMEM),
            pl.BlockSpec(memory_space=pltpu.VMEM),
            pl.BlockSpec(memory_space=pltpu.VMEM),
            pl.BlockSpec(memory_space=pltpu.VMEM),
        ],
        out_specs=pl.BlockSpec(memory_space=pltpu.VMEM),
        scratch_shapes=[
            pltpu.VMEM((N_TOK, D_HID), jnp.bfloat16),
            pltpu.VMEM((N_SLOTS, CHUNK, D_HID), jnp.bfloat16),
            pltpu.SemaphoreType.DMA((N_SLOTS,)),
            pltpu.SemaphoreType.DMA((N_SLOTS,)),
        ],
        compiler_params=pltpu.CompilerParams(collective_id=0),
    )(x, router_W, route_idx, expert_W)


# device time: 120166 ns/iter; 1.0793x vs baseline; 1.0793x over previous
import jax
import jax.numpy as jnp
from jax import lax
from jax.experimental import pallas as pl
from jax.experimental.pallas import tpu as pltpu

N_DEV = 8
N_TOK = 2048
D_MODEL = 512
D_HID = 1024
N_EXP = 32
N_EXP_LOCAL = N_EXP // N_DEV
CHUNK = N_TOK // N_DEV
SPLIT = 2
SUB = CHUNK // SPLIT
N_SLOTS = 2 * (N_DEV - 1) + 1


def kernel(x, router_W, route_idx, expert_W):
    def body(x_ref, rw_ref, idx_ref, ew_ref, out_ref, scores_ref, cb_ref,
             comm_ref, send_sems, recv_sems):
        my = lax.axis_index("i")
        left = lax.rem(my + N_DEV - 1, N_DEV)
        right = lax.rem(my + 1, N_DEV)

        barrier_sem = pltpu.get_barrier_semaphore()
        for nbr in (left, right):
            pl.semaphore_signal(
                barrier_sem, inc=1,
                device_id=(nbr,), device_id_type=pl.DeviceIdType.MESH,
            )
        pl.semaphore_wait(barrier_sem, 2)

        scores_ref[:, :] = jnp.dot(x_ref[:, :], rw_ref[:, :],
                                   preferred_element_type=jnp.float32)
        ewb = ew_ref[:, :, :].astype(jnp.bfloat16)

        def compute_chunk(c):
            r0 = c * CHUNK
            sc = scores_ref[pl.ds(r0, CHUNK), :]
            idxc = idx_ref[pl.ds(r0, CHUNK), :]
            e0 = idxc[:, 0:1]
            e1 = idxc[:, 1:2]
            iota = lax.broadcasted_iota(jnp.int32, (CHUNK, N_EXP), 1)
            s0 = jnp.sum(jnp.where(iota == e0, sc, 0.0), axis=1, keepdims=True)
            s1 = jnp.sum(jnp.where(iota == e1, sc, 0.0), axis=1, keepdims=True)
            m = jnp.maximum(s0, s1)
            g0 = jnp.exp(s0 - m)
            g1 = jnp.exp(s1 - m)
            w0 = g0 / (g0 + g1)
            w1 = g1 / (g0 + g1)
            xc = x_ref[pl.ds(r0, CHUNK), :].astype(jnp.bfloat16)
            acc = jnp.zeros((CHUNK, D_HID), jnp.float32)
            for k in range(N_EXP_LOCAL):
                ge = my * N_EXP_LOCAL + k
                gate = (w0 * (e0 == ge).astype(jnp.float32)
                        + w1 * (e1 == ge).astype(jnp.float32))
                acc = acc + gate * jnp.dot(xc, ewb[k],
                                           preferred_element_type=jnp.float32)
            cb_ref[:, :] = acc.astype(jnp.bfloat16)

        def hop(j, s):
            return pltpu.make_async_remote_copy(
                src_ref=comm_ref.at[j, s],
                dst_ref=comm_ref.at[j, s + 1],
                send_sem=send_sems.at[j, s],
                recv_sem=recv_sems.at[j, s + 1],
                device_id=(right,),
                device_id_type=pl.DeviceIdType.MESH,
            )

        compute_chunk(my)
        for j in range(SPLIT):
            comm_ref[j, 0] = cb_ref[pl.ds(j * SUB, SUB), :]
        for s in range(N_DEV - 1):
            rdmas = [hop(j, s) for j in range(SPLIT)]
            for r in rdmas:
                r.start()
            compute_chunk(lax.rem(my + N_DEV - s - 1, N_DEV))
            for j in range(SPLIT):
                rdmas[j].wait_recv()
                comm_ref[j, s + 1] = (
                    comm_ref[j, s + 1][:, :].astype(jnp.float32)
                    + cb_ref[pl.ds(j * SUB, SUB), :].astype(jnp.float32)
                ).astype(jnp.bfloat16)

        for s in range(N_DEV - 1):
            slot = (N_DEV - 1) + s
            rdmas = [hop(j, slot) for j in range(SPLIT)]
            for r in rdmas:
                r.start()
            c = lax.rem(my + N_DEV + 1 - s, N_DEV)
            for j in range(SPLIT):
                out_ref[pl.ds(c * CHUNK + j * SUB, SUB), :] = comm_ref[j, slot][:, :]
            for r in rdmas:
                r.wait_recv()
        c = lax.rem(my + N_DEV - 6, N_DEV)
        for j in range(SPLIT):
            out_ref[pl.ds(c * CHUNK + j * SUB, SUB), :] = comm_ref[j, 2 * N_DEV - 2][:, :]

        for s in range(2 * (N_DEV - 1)):
            for j in range(SPLIT):
                hop(j, s).wait_send()

    return pl.pallas_call(
        body,
        out_shape=jax.ShapeDtypeStruct((N_TOK, D_HID), jnp.bfloat16),
        in_specs=[
            pl.BlockSpec(memory_space=pltpu.VMEM),
            pl.BlockSpec(memory_space=pltpu.VMEM),
            pl.BlockSpec(memory_space=pltpu.VMEM),
            pl.BlockSpec(memory_space=pltpu.VMEM),
        ],
        out_specs=pl.BlockSpec(memory_space=pltpu.VMEM),
        scratch_shapes=[
            pltpu.VMEM((N_TOK, N_EXP), jnp.float32),
            pltpu.VMEM((CHUNK, D_HID), jnp.bfloat16),
            pltpu.VMEM((SPLIT, N_SLOTS, SUB, D_HID), jnp.bfloat16),
            pltpu.SemaphoreType.DMA((SPLIT, N_SLOTS)),
            pltpu.SemaphoreType.DMA((SPLIT, N_SLOTS)),
        ],
        compiler_params=pltpu.CompilerParams(collective_id=0),
    )(x, router_W, route_idx, expert_W)


# device time: 98557 ns/iter; 1.3160x vs baseline; 1.2193x over previous
import jax
import jax.numpy as jnp
from jax import lax
from jax.experimental import pallas as pl
from jax.experimental.pallas import tpu as pltpu

N_DEV = 8
N_TOK = 2048
D_MODEL = 512
D_HID = 1024
N_EXP = 32
N_EXP_LOCAL = N_EXP // N_DEV
CHUNK = N_TOK // N_DEV
SPLIT = 2
SUB = CHUNK // SPLIT
N_SLOTS = 2 * (N_DEV - 1) + 1


def kernel(x, router_W, route_idx, expert_W):
    def body(x_ref, rw_ref, idx_ref, ew_ref, out_ref, scores_ref, cb_ref,
             comm_ref, send_sems, recv_sems):
        my = lax.axis_index("i")
        left = lax.rem(my + N_DEV - 1, N_DEV)
        right = lax.rem(my + 1, N_DEV)

        barrier_sem = pltpu.get_barrier_semaphore()
        for nbr in (left, right):
            pl.semaphore_signal(
                barrier_sem, inc=1,
                device_id=(nbr,), device_id_type=pl.DeviceIdType.MESH,
            )
        pl.semaphore_wait(barrier_sem, 2)

        scores_ref[:, :] = jnp.dot(x_ref[:, :], rw_ref[:, :],
                                   preferred_element_type=jnp.float32)
        ewb = ew_ref[:, :, :].astype(jnp.bfloat16)

        def compute_chunk(c):
            r0 = c * CHUNK
            sc = scores_ref[pl.ds(r0, CHUNK), :]
            idxc = idx_ref[pl.ds(r0, CHUNK), :]
            e0 = idxc[:, 0:1]
            e1 = idxc[:, 1:2]
            iota = lax.broadcasted_iota(jnp.int32, (CHUNK, N_EXP), 1)
            s0 = jnp.sum(jnp.where(iota == e0, sc, 0.0), axis=1, keepdims=True)
            s1 = jnp.sum(jnp.where(iota == e1, sc, 0.0), axis=1, keepdims=True)
            m = jnp.maximum(s0, s1)
            g0 = jnp.exp(s0 - m)
            g1 = jnp.exp(s1 - m)
            w0 = g0 / (g0 + g1)
            w1 = g1 / (g0 + g1)
            xc = x_ref[pl.ds(r0, CHUNK), :].astype(jnp.bfloat16)
            acc = jnp.zeros((CHUNK, D_HID), jnp.float32)
            for k in range(N_EXP_LOCAL):
                ge = my * N_EXP_LOCAL + k
                gate = (w0 * (e0 == ge).astype(jnp.float32)
                        + w1 * (e1 == ge).astype(jnp.float32))
                acc = acc + gate * jnp.dot(xc, ewb[k],
                                           preferred_element_type=jnp.float32)
            cb_ref[:, :] = acc.astype(jnp.bfloat16)

        def hop(j, s):
            return pltpu.make_async_remote_copy(
                src_ref=comm_ref.at[j, s],
                dst_ref=comm_ref.at[j, s + 1],
                send_sem=send_sems.at[j, s],
                recv_sem=recv_sems.at[j, s + 1],
                device_id=(right,),
                device_id_type=pl.DeviceIdType.MESH,
            )

        compute_chunk(my)
        for j in range(SPLIT):
            comm_ref[j, 0] = cb_ref[pl.ds(j * SUB, SUB), :]
        cur = [hop(j, 0) for j in range(SPLIT)]
        for r in cur:
            r.start()
        compute_chunk(lax.rem(my + N_DEV - 1, N_DEV))
        for s in range(N_DEV - 1):
            nxt = [hop(j, s + 1) for j in range(SPLIT)] if s < N_DEV - 2 else None
            for j in range(SPLIT):
                cur[j].wait_recv()
                comm_ref[j, s + 1] = (comm_ref[j, s + 1][:, :]
                                      + cb_ref[pl.ds(j * SUB, SUB), :])
                if nxt is not None:
                    nxt[j].start()
            if s < N_DEV - 2:
                compute_chunk(lax.rem(my + N_DEV - s - 2, N_DEV))
                cur = nxt

        cur = [hop(j, N_DEV - 1) for j in range(SPLIT)]
        for r in cur:
            r.start()
        own = lax.rem(my + 1, N_DEV)
        for j in range(SPLIT):
            out_ref[pl.ds(own * CHUNK + j * SUB, SUB), :] = (
                comm_ref[j, N_DEV - 1][:, :])
        for s in range(N_DEV - 1):
            slot = N_DEV + s
            nxt = [hop(j, slot) for j in range(SPLIT)] if s < N_DEV - 2 else None
            c = lax.rem(my + N_DEV - s, N_DEV)
            for j in range(SPLIT):
                cur[j].wait_recv()
                if nxt is not None:
                    nxt[j].start()
                out_ref[pl.ds(c * CHUNK + j * SUB, SUB), :] = (
                    comm_ref[j, slot][:, :])
            if nxt is not None:
                cur = nxt

        for s in range(2 * (N_DEV - 1)):
            for j in range(SPLIT):
                hop(j, s).wait_send()

    return pl.pallas_call(
        body,
        out_shape=jax.ShapeDtypeStruct((N_TOK, D_HID), jnp.bfloat16),
        in_specs=[
            pl.BlockSpec(memory_space=pltpu.VMEM),
            pl.BlockSpec(memory_space=pltpu.VMEM),
            pl.BlockSpec(memory_space=pltpu.VMEM),
            pl.BlockSpec(memory_space=pltpu.VMEM),
        ],
        out_specs=pl.BlockSpec(memory_space=pltpu.VMEM),
        scratch_shapes=[
            pltpu.VMEM((N_TOK, N_EXP), jnp.float32),
            pltpu.VMEM((CHUNK, D_HID), jnp.bfloat16),
            pltpu.VMEM((SPLIT, N_SLOTS, SUB, D_HID), jnp.bfloat16),
            pltpu.SemaphoreType.DMA((SPLIT, N_SLOTS)),
            pltpu.SemaphoreType.DMA((SPLIT, N_SLOTS)),
        ],
        compiler_params=pltpu.CompilerParams(collective_id=0),
    )(x, router_W, route_idx, expert_W)


# device time: 81356 ns/iter; 1.5942x vs baseline; 1.2114x over previous
import jax
import jax.numpy as jnp
from jax import lax
from jax.experimental import pallas as pl
from jax.experimental.pallas import tpu as pltpu

N_DEV = 8
N_TOK = 2048
D_MODEL = 512
D_HID = 1024
N_EXP = 32
N_EXP_LOCAL = N_EXP // N_DEV
CHUNK = N_TOK // N_DEV
SPLIT = 2
SUB = CHUNK // SPLIT
N_SLOTS = 2 * (N_DEV - 1) + 1


def kernel(x, router_W, route_idx, expert_W):
    def body(x_ref, rw_ref, idx_ref, ew_ref, out_ref, scores_ref, cb_ref,
             comm_ref, send_sems, recv_sems):
        my = lax.axis_index("i")
        left = lax.rem(my + N_DEV - 1, N_DEV)
        right = lax.rem(my + 1, N_DEV)

        barrier_sem = pltpu.get_barrier_semaphore()
        for nbr in (left, right):
            pl.semaphore_signal(
                barrier_sem, inc=1,
                device_id=(nbr,), device_id_type=pl.DeviceIdType.MESH,
            )
        pl.semaphore_wait(barrier_sem, 2)

        scores_ref[:, :] = jnp.dot(x_ref[:, :], rw_ref[:, :],
                                   preferred_element_type=jnp.float32)
        ewb = ew_ref[:, :, :].astype(jnp.bfloat16)

        def compute_sub(c, j):
            r0 = c * CHUNK + j * SUB
            sc = scores_ref[pl.ds(r0, SUB), :]
            idxc = idx_ref[pl.ds(r0, SUB), :]
            e0 = idxc[:, 0:1]
            e1 = idxc[:, 1:2]
            iota = lax.broadcasted_iota(jnp.int32, (SUB, N_EXP), 1)
            s0 = jnp.sum(jnp.where(iota == e0, sc, 0.0), axis=1, keepdims=True)
            s1 = jnp.sum(jnp.where(iota == e1, sc, 0.0), axis=1, keepdims=True)
            m = jnp.maximum(s0, s1)
            g0 = jnp.exp(s0 - m)
            g1 = jnp.exp(s1 - m)
            w0 = g0 / (g0 + g1)
            w1 = g1 / (g0 + g1)
            xc = x_ref[pl.ds(r0, SUB), :].astype(jnp.bfloat16)
            acc = jnp.zeros((SUB, D_HID), jnp.float32)
            for k in range(N_EXP_LOCAL):
                ge = my * N_EXP_LOCAL + k
                gate = (w0 * (e0 == ge).astype(jnp.float32)
                        + w1 * (e1 == ge).astype(jnp.float32))
                acc = acc + gate * jnp.dot(xc, ewb[k],
                                           preferred_element_type=jnp.float32)
            cb_ref[pl.ds(j * SUB, SUB), :] = acc.astype(jnp.bfloat16)

        def hop(j, s):
            return pltpu.make_async_remote_copy(
                src_ref=comm_ref.at[j, s],
                dst_ref=comm_ref.at[j, s + 1],
                send_sem=send_sems.at[j, s],
                recv_sem=recv_sems.at[j, s + 1],
                device_id=(right,) if j == 0 else (left,),
                device_id_type=pl.DeviceIdType.MESH,
            )

        rs_chunk = (
            lambda s: lax.rem(my + N_DEV - s - 1, N_DEV),
            lambda s: lax.rem(my + s + 1, N_DEV),
        )
        for j in range(SPLIT):
            compute_sub(my, j)
            comm_ref[j, 0] = cb_ref[pl.ds(j * SUB, SUB), :]
        cur = [hop(j, 0) for j in range(SPLIT)]
        for r in cur:
            r.start()
        for j in range(SPLIT):
            compute_sub(rs_chunk[j](0), j)
        for s in range(N_DEV - 1):
            nxt = [hop(j, s + 1) for j in range(SPLIT)] if s < N_DEV - 2 else None
            for j in range(SPLIT):
                cur[j].wait_recv()
                comm_ref[j, s + 1] = (comm_ref[j, s + 1][:, :]
                                      + cb_ref[pl.ds(j * SUB, SUB), :])
                if nxt is not None:
                    nxt[j].start()
            if s < N_DEV - 2:
                for j in range(SPLIT):
                    compute_sub(rs_chunk[j](s + 1), j)
                cur = nxt

        ag_chunk = (
            lambda s: lax.rem(my + N_DEV - s, N_DEV),
            lambda s: lax.rem(my + s, N_DEV),
        )
        cur = [hop(j, N_DEV - 1) for j in range(SPLIT)]
        for r in cur:
            r.start()
        own = (lax.rem(my + 1, N_DEV), lax.rem(my + N_DEV - 1, N_DEV))
        for j in range(SPLIT):
            out_ref[pl.ds(own[j] * CHUNK + j * SUB, SUB), :] = (
                comm_ref[j, N_DEV - 1][:, :])
        for s in range(N_DEV - 1):
            slot = N_DEV + s
            nxt = [hop(j, slot) for j in range(SPLIT)] if s < N_DEV - 2 else None
            for j in range(SPLIT):
                cur[j].wait_recv()
                if nxt is not None:
                    nxt[j].start()
                out_ref[pl.ds(ag_chunk[j](s) * CHUNK + j * SUB, SUB), :] = (
                    comm_ref[j, slot][:, :])
            if nxt is not None:
                cur = nxt

        for s in range(2 * (N_DEV - 1)):
            for j in range(SPLIT):
                hop(j, s).wait_send()

    return pl.pallas_call(
        body,
        out_shape=jax.ShapeDtypeStruct((N_TOK, D_HID), jnp.bfloat16),
        in_specs=[
            pl.BlockSpec(memory_space=pltpu.VMEM),
            pl.BlockSpec(memory_space=pltpu.VMEM),
            pl.BlockSpec(memory_space=pltpu.VMEM),
            pl.BlockSpec(memory_space=pltpu.VMEM),
        ],
        out_specs=pl.BlockSpec(memory_space=pltpu.VMEM),
        scratch_shapes=[
            pltpu.VMEM((N_TOK, N_EXP), jnp.float32),
            pltpu.VMEM((CHUNK, D_HID), jnp.bfloat16),
            pltpu.VMEM((SPLIT, N_SLOTS, SUB, D_HID), jnp.bfloat16),
            pltpu.SemaphoreType.DMA((SPLIT, N_SLOTS)),
            pltpu.SemaphoreType.DMA((SPLIT, N_SLOTS)),
        ],
        compiler_params=pltpu.CompilerParams(collective_id=0),
    )(x, router_W, route_idx, expert_W)


# device time: 65688 ns/iter; 1.9745x vs baseline; 1.2385x over previous
import jax
import jax.numpy as jnp
from jax import lax
from jax.experimental import pallas as pl
from jax.experimental.pallas import tpu as pltpu

N_DEV = 8
N_TOK = 2048
D_MODEL = 512
D_HID = 1024
N_EXP = 32
N_EXP_LOCAL = N_EXP // N_DEV
CHUNK = N_TOK // N_DEV
HALF = CHUNK // 2
DIR_SPLIT = 2
SUBH = HALF // DIR_SPLIT
SUBRINGS = [(d, h) for h in range(DIR_SPLIT) for d in (0, 1)]
N_SLOTS = 2 * (N_DEV - 1) + 1


def kernel(x, router_W, route_idx, expert_W):
    def body(x_ref, rw_ref, idx_ref, ew_ref, out_ref, scores_ref, cb_ref,
             comm_ref, send_sems, recv_sems):
        my = lax.axis_index("i")
        left = lax.rem(my + N_DEV - 1, N_DEV)
        right = lax.rem(my + 1, N_DEV)

        barrier_sem = pltpu.get_barrier_semaphore()
        for nbr in (left, right):
            pl.semaphore_signal(
                barrier_sem, inc=1,
                device_id=(nbr,), device_id_type=pl.DeviceIdType.MESH,
            )
        pl.semaphore_wait(barrier_sem, 2)

        scores_ref[:, :] = jnp.dot(x_ref[:, :], rw_ref[:, :],
                                   preferred_element_type=jnp.float32)
        ewb = ew_ref[:, :, :].astype(jnp.bfloat16)

        def compute_half(c, d):
            r0 = c * CHUNK + d * HALF
            sc = scores_ref[pl.ds(r0, HALF), :]
            idxc = idx_ref[pl.ds(r0, HALF), :]
            e0 = idxc[:, 0:1]
            e1 = idxc[:, 1:2]
            iota = lax.broadcasted_iota(jnp.int32, (HALF, N_EXP), 1)
            s0 = jnp.sum(jnp.where(iota == e0, sc, 0.0), axis=1, keepdims=True)
            s1 = jnp.sum(jnp.where(iota == e1, sc, 0.0), axis=1, keepdims=True)
            m = jnp.maximum(s0, s1)
            g0 = jnp.exp(s0 - m)
            g1 = jnp.exp(s1 - m)
            w0 = g0 / (g0 + g1)
            w1 = g1 / (g0 + g1)
            xc = x_ref[pl.ds(r0, HALF), :].astype(jnp.bfloat16)
            acc = jnp.zeros((HALF, D_HID), jnp.float32)
            for k in range(N_EXP_LOCAL):
                ge = my * N_EXP_LOCAL + k
                gate = (w0 * (e0 == ge).astype(jnp.float32)
                        + w1 * (e1 == ge).astype(jnp.float32))
                acc = acc + gate * jnp.dot(xc, ewb[k],
                                           preferred_element_type=jnp.float32)
            cb_ref[pl.ds(d * HALF, HALF), :] = acc.astype(jnp.bfloat16)

        def cb_piece(d, h):
            return pl.ds(d * HALF + h * SUBH, SUBH)

        def hop(i, s):
            d, _ = SUBRINGS[i]
            return pltpu.make_async_remote_copy(
                src_ref=comm_ref.at[i, s],
                dst_ref=comm_ref.at[i, s + 1],
                send_sem=send_sems.at[i, s],
                recv_sem=recv_sems.at[i, s + 1],
                device_id=(right,) if d == 0 else (left,),
                device_id_type=pl.DeviceIdType.MESH,
            )

        rs_chunk = (
            lambda s: lax.rem(my + N_DEV - s - 1, N_DEV),
            lambda s: lax.rem(my + s + 1, N_DEV),
        )

        for d in (0, 1):
            compute_half(my, d)
        for i, (d, h) in enumerate(SUBRINGS):
            comm_ref[i, 0] = cb_ref[cb_piece(d, h), :]
        cur = [hop(i, 0) for i in range(len(SUBRINGS))]
        for r in cur:
            r.start()
        for d in (0, 1):
            compute_half(rs_chunk[d](0), d)
        for s in range(N_DEV - 1):
            last = s == N_DEV - 2
            nxt = None if last else [hop(i, s + 1) for i in range(len(SUBRINGS))]
            for i, (d, h) in enumerate(SUBRINGS):
                cur[i].wait_recv()
                comm_ref[i, s + 1] = (comm_ref[i, s + 1][:, :]
                                      + cb_ref[cb_piece(d, h), :])
                if nxt is not None:
                    nxt[i].start()
            if nxt is not None:
                for d in (0, 1):
                    compute_half(rs_chunk[d](s + 1), d)
                cur = nxt

        ag_chunk = (
            lambda s: lax.rem(my + N_DEV - s, N_DEV),
            lambda s: lax.rem(my + s, N_DEV),
        )
        own = (lax.rem(my + 1, N_DEV), lax.rem(my + N_DEV - 1, N_DEV))
        cur = [hop(i, N_DEV - 1) for i in range(len(SUBRINGS))]
        for r in cur:
            r.start()
        for i, (d, h) in enumerate(SUBRINGS):
            out_ref[pl.ds(own[d] * CHUNK + d * HALF + h * SUBH, SUBH), :] = (
                comm_ref[i, N_DEV - 1][:, :])
        for s in range(N_DEV - 1):
            slot = N_DEV + s
            last = s == N_DEV - 2
            nxt = None if last else [hop(i, slot) for i in range(len(SUBRINGS))]
            for i, (d, h) in enumerate(SUBRINGS):
                cur[i].wait_recv()
                if nxt is not None:
                    nxt[i].start()
                out_ref[pl.ds(ag_chunk[d](s) * CHUNK + d * HALF + h * SUBH,
                              SUBH), :] = comm_ref[i, slot][:, :]
            if nxt is not None:
                cur = nxt

        for s in range(2 * (N_DEV - 1)):
            for i in range(len(SUBRINGS)):
                hop(i, s).wait_send()

    return pl.pallas_call(
        body,
        out_shape=jax.ShapeDtypeStruct((N_TOK, D_HID), jnp.bfloat16),
        in_specs=[
            pl.BlockSpec(memory_space=pltpu.VMEM),
            pl.BlockSpec(memory_space=pltpu.VMEM),
            pl.BlockSpec(memory_space=pltpu.VMEM),
            pl.BlockSpec(memory_space=pltpu.VMEM),
        ],
        out_specs=pl.BlockSpec(memory_space=pltpu.VMEM),
        scratch_shapes=[
            pltpu.VMEM((N_TOK, N_EXP), jnp.float32),
            pltpu.VMEM((CHUNK, D_HID), jnp.bfloat16),
            pltpu.VMEM((len(SUBRINGS), N_SLOTS, SUBH, D_HID),
                       jnp.bfloat16),
            pltpu.SemaphoreType.DMA((len(SUBRINGS), N_SLOTS)),
            pltpu.SemaphoreType.DMA((len(SUBRINGS), N_SLOTS)),
        ],
        compiler_params=pltpu.CompilerParams(collective_id=0),
    )(x, router_W, route_idx, expert_W)


# device time: 64216 ns/iter; 2.0198x vs baseline; 1.0229x over previous
import jax
import jax.numpy as jnp
from jax import lax
from jax.experimental import pallas as pl
from jax.experimental.pallas import tpu as pltpu

N_DEV = 8
N_TOK = 2048
D_MODEL = 512
D_HID = 1024
N_EXP = 32
N_EXP_LOCAL = N_EXP // N_DEV
CHUNK = N_TOK // N_DEV
HALF = CHUNK // 2
DIR_SPLIT = 4
SUBH = HALF // DIR_SPLIT
SUBRINGS = [(d, h) for h in range(DIR_SPLIT) for d in (0, 1)]
N_SLOTS = 2 * (N_DEV - 1) + 1


def kernel(x, router_W, route_idx, expert_W):
    def body(x_ref, rw_ref, idx_ref, ew_ref, out_ref, scores_ref, cb_ref,
             comm_ref, send_sems, recv_sems):
        my = lax.axis_index("i")
        left = lax.rem(my + N_DEV - 1, N_DEV)
        right = lax.rem(my + 1, N_DEV)

        barrier_sem = pltpu.get_barrier_semaphore()
        for nbr in (left, right):
            pl.semaphore_signal(
                barrier_sem, inc=1,
                device_id=(nbr,), device_id_type=pl.DeviceIdType.MESH,
            )
        pl.semaphore_wait(barrier_sem, 2)

        scores_ref[:, :] = jnp.dot(x_ref[:, :], rw_ref[:, :],
                                   preferred_element_type=jnp.float32)
        ewb = ew_ref[:, :, :].astype(jnp.bfloat16)

        def compute_half(c, d):
            r0 = c * CHUNK + d * HALF
            sc = scores_ref[pl.ds(r0, HALF), :]
            idxc = idx_ref[pl.ds(r0, HALF), :]
            e0 = idxc[:, 0:1]
            e1 = idxc[:, 1:2]
            iota = lax.broadcasted_iota(jnp.int32, (HALF, N_EXP), 1)
            s0 = jnp.sum(jnp.where(iota == e0, sc, 0.0), axis=1, keepdims=True)
            s1 = jnp.sum(jnp.where(iota == e1, sc, 0.0), axis=1, keepdims=True)
            m = jnp.maximum(s0, s1)
            g0 = jnp.exp(s0 - m)
            g1 = jnp.exp(s1 - m)
            w0 = g0 / (g0 + g1)
            w1 = g1 / (g0 + g1)
            xc = x_ref[pl.ds(r0, HALF), :].astype(jnp.bfloat16)
            acc = jnp.zeros((HALF, D_HID), jnp.float32)
            for k in range(N_EXP_LOCAL):
                ge = my * N_EXP_LOCAL + k
                gate = (w0 * (e0 == ge).astype(jnp.float32)
                        + w1 * (e1 == ge).astype(jnp.float32))
                acc = acc + gate * jnp.dot(xc, ewb[k],
                                           preferred_element_type=jnp.float32)
            cb_ref[pl.ds(d * HALF, HALF), :] = acc.astype(jnp.bfloat16)

        def cb_piece(d, h):
            return pl.ds(d * HALF + h * SUBH, SUBH)

        def hop(i, s):
            d, _ = SUBRINGS[i]
            return pltpu.make_async_remote_copy(
                src_ref=comm_ref.at[i, s],
                dst_ref=comm_ref.at[i, s + 1],
                send_sem=send_sems.at[i, s],
                recv_sem=recv_sems.at[i, s + 1],
                device_id=(right,) if d == 0 else (left,),
                device_id_type=pl.DeviceIdType.MESH,
            )

        rs_chunk = (
            lambda s: lax.rem(my + N_DEV - s - 1, N_DEV),
            lambda s: lax.rem(my + s + 1, N_DEV),
        )

        cur = [None] * len(SUBRINGS)
        for d in (0, 1):
            compute_half(my, d)
            for i, (dd, h) in enumerate(SUBRINGS):
                if dd == d:
                    comm_ref[i, 0] = cb_ref[cb_piece(d, h), :]
                    cur[i] = hop(i, 0)
                    cur[i].start()
        for d in (0, 1):
            compute_half(rs_chunk[d](0), d)
        for s in range(N_DEV - 1):
            last = s == N_DEV - 2
            nxt = None if last else [hop(i, s + 1) for i in range(len(SUBRINGS))]
            for i, (d, h) in enumerate(SUBRINGS):
                cur[i].wait_recv()
                comm_ref[i, s + 1] = (comm_ref[i, s + 1][:, :]
                                      + cb_ref[cb_piece(d, h), :])
                if nxt is not None:
                    nxt[i].start()
            if nxt is not None:
                for d in (0, 1):
                    compute_half(rs_chunk[d](s + 1), d)
                cur = nxt

        ag_chunk = (
            lambda s: lax.rem(my + N_DEV - s, N_DEV),
            lambda s: lax.rem(my + s, N_DEV),
        )
        own = (lax.rem(my + 1, N_DEV), lax.rem(my + N_DEV - 1, N_DEV))
        cur = [hop(i, N_DEV - 1) for i in range(len(SUBRINGS))]
        for r in cur:
            r.start()
        for i, (d, h) in enumerate(SUBRINGS):
            out_ref[pl.ds(own[d] * CHUNK + d * HALF + h * SUBH, SUBH), :] = (
                comm_ref[i, N_DEV - 1][:, :])
        for s in range(N_DEV - 1):
            slot = N_DEV + s
            last = s == N_DEV - 2
            nxt = None if last else [hop(i, slot) for i in range(len(SUBRINGS))]
            for i, (d, h) in enumerate(SUBRINGS):
                cur[i].wait_recv()
                if nxt is not None:
                    nxt[i].start()
                out_ref[pl.ds(ag_chunk[d](s) * CHUNK + d * HALF + h * SUBH,
                              SUBH), :] = comm_ref[i, slot][:, :]
            if nxt is not None:
                cur = nxt

        for s in range(2 * (N_DEV - 1)):
            for i in range(len(SUBRINGS)):
                hop(i, s).wait_send()

    return pl.pallas_call(
        body,
        out_shape=jax.ShapeDtypeStruct((N_TOK, D_HID), jnp.bfloat16),
        in_specs=[
            pl.BlockSpec(memory_space=pltpu.VMEM),
            pl.BlockSpec(memory_space=pltpu.VMEM),
            pl.BlockSpec(memory_space=pltpu.VMEM),
            pl.BlockSpec(memory_space=pltpu.VMEM),
        ],
        out_specs=pl.BlockSpec(memory_space=pltpu.VMEM),
        scratch_shapes=[
            pltpu.VMEM((N_TOK, N_EXP), jnp.float32),
            pltpu.VMEM((CHUNK, D_HID), jnp.bfloat16),
            pltpu.VMEM((len(SUBRINGS), N_SLOTS, SUBH, D_HID),
                       jnp.bfloat16),
            pltpu.SemaphoreType.DMA((len(SUBRINGS), N_SLOTS)),
            pltpu.SemaphoreType.DMA((len(SUBRINGS), N_SLOTS)),
        ],
        compiler_params=pltpu.CompilerParams(collective_id=0),
    )(x, router_W, route_idx, expert_W)
